# Initial kernel scaffold; baseline (speedup 1.0000x reference)
import functools

import jax
import jax.numpy as jnp
from jax import lax
from jax.experimental import pallas as pl
from jax.experimental.pallas import tpu as pltpu
from jax.experimental.pallas import tpu_sc as plsc

ROWS = 160000
D = 256
HALF = 128
S = 10000

NC = 2
NS = 16
CHUNK = 80
ROWS_PER_TILE = ROWS // NS
CHUNKS = ROWS_PER_TILE // CHUNK
SEGS_PER_TILE = S // NS
SEG_SUB = 5


def _body(x_hbm, ids_hbm, out_hbm,
          ids_v, buf0, buf1, sums_buf, cnts_buf, ones_v,
          acc_sh, cnt_sh, sem0, sem1):
  c = lax.axis_index("c")
  t = lax.axis_index("s")
  col0 = c * HALF
  row0 = t * ROWS_PER_TILE
  seg0 = t * SEGS_PER_TILE

  ones16 = jnp.ones((16,), jnp.float32)
  zeros16 = jnp.zeros((16,), jnp.float32)

  def fill(i, _):
    ones_v[i % CHUNK, :] = ones16
    for k in range(HALF // 16):
      sums_buf[i, pl.ds(k * 16, 16)] = zeros16
    cnts_buf[i, :] = zeros16
    return 0
  lax.fori_loop(0, 125, fill, 0)

  for sub in range(SEG_SUB):
    pltpu.sync_copy(sums_buf, acc_sh.at[pl.ds(seg0 + sub * 125, 125), :])
    pltpu.sync_copy(cnts_buf, cnt_sh.at[pl.ds(seg0 + sub * 125, 125), :])
  plsc.subcore_barrier()

  pltpu.sync_copy(ids_hbm.at[pl.ds(t * CHUNKS, CHUNKS), :], ids_v)

  def x_slice(j):
    return x_hbm.at[pl.ds(row0 + j * CHUNK, CHUNK), pl.ds(col0, HALF)]

  def start_load(j, buf, sem):
    pltpu.async_copy(x_slice(j), buf, sem)

  def wait_load(j, buf, sem):
    pltpu.make_async_copy(x_slice(j), buf, sem).wait()

  def scatter(j, buf):
    idx = ids_v.at[j]
    pltpu.sync_copy(buf, acc_sh.at[idx], add=True)
    pltpu.sync_copy(ones_v, cnt_sh.at[idx], add=True)

  start_load(0, buf0, sem0)

  def step(gp, _):
    j0 = 2 * gp
    wait_load(j0, buf0, sem0)
    start_load(j0 + 1, buf1, sem1)
    scatter(j0, buf0)
    wait_load(j0 + 1, buf1, sem1)
    start_load(j0 + 2, buf0, sem0)
    scatter(j0 + 1, buf1)
    return 0
  lax.fori_loop(0, (CHUNKS - 1) // 2, step, 0)

  wait_load(CHUNKS - 1, buf0, sem0)
  scatter(CHUNKS - 1, buf0)
  plsc.subcore_barrier()

  for sub in range(SEG_SUB):
    segbase = seg0 + sub * 125
    pltpu.sync_copy(acc_sh.at[pl.ds(segbase, 125), :], sums_buf)
    pltpu.sync_copy(cnt_sh.at[pl.ds(segbase, 125), :], cnts_buf)

    def per_seg(i, _):
      cvec = cnts_buf[i, :]
      r = 1.0 / jnp.maximum(cvec, 1.0)
      for k in range(HALF // 16):
        sums_buf[i, pl.ds(k * 16, 16)] = sums_buf[i, pl.ds(k * 16, 16)] * r
      return 0
    lax.fori_loop(0, 125, per_seg, 0)

    pltpu.sync_copy(sums_buf, out_hbm.at[pl.ds(segbase, 125), pl.ds(col0, HALF)])


def _build():
  return pl.kernel(
      _body,
      out_type=jax.ShapeDtypeStruct((S, D), jnp.float32),
      mesh=plsc.VectorSubcoreMesh(
          core_axis_name="c", subcore_axis_name="s",
          num_cores=NC, num_subcores=NS),
      scratch_types=[
          pltpu.VMEM((CHUNKS, CHUNK), jnp.int32),
          pltpu.VMEM((CHUNK, HALF), jnp.float32),
          pltpu.VMEM((CHUNK, HALF), jnp.float32),
          pltpu.VMEM((125, HALF), jnp.float32),
          pltpu.VMEM((125, 16), jnp.float32),
          pltpu.VMEM((CHUNK, 16), jnp.float32),
          pltpu.VMEM_SHARED((S, HALF), jnp.float32),
          pltpu.VMEM_SHARED((S, 16), jnp.float32),
          pltpu.SemaphoreType.DMA,
          pltpu.SemaphoreType.DMA,
      ],
  )


@jax.jit
def kernel(x, segment_ids):
  ids = jnp.asarray(segment_ids, jnp.int32).reshape(ROWS // CHUNK, CHUNK)
  return _build()(x, ids)


# speedup vs baseline: 2.6148x; 2.6148x over previous
import jax
import jax.numpy as jnp
from jax import lax
from jax.experimental import pallas as pl
from jax.experimental.pallas import tpu as pltpu
from jax.experimental.pallas import tpu_sc as plsc

ROWS = 160000
D = 256
HALF = 128
QW = 64
S = 10000
S_PAD = 10240

NC = 2
NS = 16
CHUNK = 80
ROWS_PER_TILE = ROWS // NS
CHUNKS = ROWS_PER_TILE // CHUNK
SEGS_PER_TILE = S_PAD // NS
SEG_SUB = SEGS_PER_TILE // 128


def _body(x_hbm, ids_hbm, out_hbm,
          ids_v, buf0, buf1, sums_buf, cnts_buf, ones_v,
          acc_sh, cnt_sh, sem0, sem1):
  c = lax.axis_index("c")
  t = lax.axis_index("s")
  row0 = t * ROWS_PER_TILE
  seg0 = t * SEGS_PER_TILE

  ones16 = jnp.ones((16,), jnp.float32)
  zeros16 = jnp.zeros((16,), jnp.float32)

  def fill(i, _):
    ones_v[i % CHUNK, :] = ones16
    for k in range(QW // 16):
      sums_buf[i, pl.ds(k * 16, 16)] = zeros16
    cnts_buf[i, :] = zeros16
    return 0
  lax.fori_loop(0, 128, fill, 0)

  def zero_acc_stripe():
    for sub in range(SEG_SUB):
      pltpu.sync_copy(sums_buf, acc_sh.at[pl.ds(seg0 + sub * 128, 128), :])

  zero_acc_stripe()
  for sub in range(SEG_SUB):
    pltpu.sync_copy(cnts_buf, cnt_sh.at[pl.ds(seg0 + sub * 128, 128), :])
  plsc.subcore_barrier()

  pltpu.sync_copy(ids_hbm.at[t], ids_v)

  for p in range(2):
    col0 = c * HALF + p * QW

    def x_slice(j):
      return x_hbm.at[pl.ds(row0 + j * CHUNK, CHUNK), pl.ds(col0, QW)]

    def start_load(j, buf, sem):
      pltpu.async_copy(x_slice(j), buf, sem)

    def wait_load(j, buf, sem):
      pltpu.make_async_copy(x_slice(j), buf, sem).wait()

    def scatter(j, buf):
      idx = ids_v.at[j]
      pltpu.sync_copy(buf, acc_sh.at[idx], add=True)
      if p == 0:
        pltpu.sync_copy(ones_v, cnt_sh.at[idx], add=True)

    start_load(0, buf0, sem0)

    def step(gp, _):
      j0 = 2 * gp
      wait_load(j0, buf0, sem0)
      start_load(j0 + 1, buf1, sem1)
      scatter(j0, buf0)
      wait_load(j0 + 1, buf1, sem1)
      start_load(j0 + 2, buf0, sem0)
      scatter(j0 + 1, buf1)
      return 0
    lax.fori_loop(0, (CHUNKS - 1) // 2, step, 0)

    wait_load(CHUNKS - 1, buf0, sem0)
    scatter(CHUNKS - 1, buf0)
    plsc.subcore_barrier()

    for sub in range(SEG_SUB):
      segbase = seg0 + sub * 128
      pltpu.sync_copy(acc_sh.at[pl.ds(segbase, 128), :], sums_buf)
      pltpu.sync_copy(cnt_sh.at[pl.ds(segbase, 128), :], cnts_buf)

      def per_seg(i, _):
        cvec = cnts_buf[i, :]
        r = 1.0 / jnp.maximum(cvec, 1.0)
        for k in range(QW // 16):
          sums_buf[i, pl.ds(k * 16, 16)] = sums_buf[i, pl.ds(k * 16, 16)] * r
        return 0
      lax.fori_loop(0, 128, per_seg, 0)

      pltpu.sync_copy(sums_buf, out_hbm.at[pl.ds(segbase, 128), pl.ds(col0, QW)])

    if p == 0:
      def zfill(i, _):
        for k in range(QW // 16):
          sums_buf[i, pl.ds(k * 16, 16)] = zeros16
        return 0
      lax.fori_loop(0, 128, zfill, 0)
      zero_acc_stripe()
      plsc.subcore_barrier()


def _build():
  return pl.kernel(
      _body,
      out_type=jax.ShapeDtypeStruct((S_PAD, D), jnp.float32),
      mesh=plsc.VectorSubcoreMesh(
          core_axis_name="c", subcore_axis_name="s",
          num_cores=NC, num_subcores=NS),
      scratch_types=[
          pltpu.VMEM((CHUNKS, CHUNK), jnp.int32),
          pltpu.VMEM((CHUNK, QW), jnp.float32),
          pltpu.VMEM((CHUNK, QW), jnp.float32),
          pltpu.VMEM((128, QW), jnp.float32),
          pltpu.VMEM((128, 16), jnp.float32),
          pltpu.VMEM((CHUNK, 16), jnp.float32),
          pltpu.VMEM_SHARED((S_PAD, QW), jnp.float32),
          pltpu.VMEM_SHARED((S_PAD, 16), jnp.float32),
          pltpu.SemaphoreType.DMA,
          pltpu.SemaphoreType.DMA,
      ],
      compiler_params=pltpu.CompilerParams(use_tc_tiling_on_sc=False),
  )


@jax.jit
def kernel(x, segment_ids):
  ids = jnp.asarray(segment_ids, jnp.int32).reshape(NS, CHUNKS, CHUNK)
  return _build()(x, ids)[:S]


# speedup vs baseline: 3.4543x; 1.3211x over previous
import jax
import jax.numpy as jnp
from jax import lax
from jax.experimental import pallas as pl
from jax.experimental.pallas import tpu as pltpu
from jax.experimental.pallas import tpu_sc as plsc

ROWS = 160000
D = 256
HALF = 128
S = 10000
S_PAD = 10240

NC = 2
NS = 16
CH = 128
NCHUNKS = ROWS // CH
NCH_PAD = 1280
BASES = (0, 3456, 6912)
SPANS = (3456, 3456, 3328)
ACC_ROWS = 3456
DUMMY = ACC_ROWS
NTPT = NCH_PAD // NS
DUMMY_ROW = NTPT


def _stripe_subs(span):
  st = span // NS
  return ((0, 128), (128, st - 128))


def _body(x_hbm, idx_hbm, bounds_hbm, out_hbm,
          bounds_v, idx_v, buf0, buf1, sums_buf, cnts_buf, ones_v,
          acc_sh, cnt_sh, sem0, sem1):
  c = lax.axis_index("c")
  t = lax.axis_index("s")
  col0 = c * HALF

  ones16 = jnp.ones((16,), jnp.float32)
  zeros16 = jnp.zeros((16,), jnp.float32)

  def fill(i, _):
    ones_v[i, :] = ones16
    for k in range(HALF // 16):
      sums_buf[i, pl.ds(k * 16, 16)] = zeros16
    cnts_buf[i, :] = zeros16
    return 0
  lax.fori_loop(0, 128, fill, 0)

  def zero_acc():
    base = t * (ACC_ROWS // NS)
    for off, sz in _stripe_subs(ACC_ROWS):
      pltpu.sync_copy(sums_buf.at[pl.ds(0, sz), :],
                      acc_sh.at[pl.ds(base + off, sz), :])
      pltpu.sync_copy(cnts_buf.at[pl.ds(0, sz), :],
                      cnt_sh.at[pl.ds(base + off, sz), :])

  zero_acc()
  pltpu.sync_copy(bounds_hbm.at[t], bounds_v)
  plsc.subcore_barrier()
  bv = bounds_v[:]
  pass_lo = (jnp.int32(0), bv[2], bv[3])
  pass_hi = (bv[0], bv[1], bv[4])

  def x_slice(jj):
    return x_hbm.at[pl.ds(t * CH + jj * (16 * CH), CH), pl.ds(col0, HALF)]

  def run_pass(p, lo, hi):
    pltpu.sync_copy(idx_hbm.at[p * NS + t], idx_v)

    def cl(j):
      return jnp.maximum(jnp.minimum(j, hi - 1), 0)

    def sel(j):
      return jnp.where(j < hi, j, DUMMY_ROW)

    def start_load(j, buf, sem):
      pltpu.async_copy(x_slice(cl(j)), buf, sem)

    def wait_load(j, buf, sem):
      pltpu.make_async_copy(x_slice(cl(j)), buf, sem).wait()

    def scatter(j, buf):
      idx = idx_v.at[sel(j)]
      pltpu.sync_copy(buf, acc_sh.at[idx], add=True)
      pltpu.sync_copy(ones_v, cnt_sh.at[idx], add=True)

    npairs = (hi - lo + 1) // 2
    start_load(lo, buf0, sem0)

    def pair(m, _):
      j0 = lo + 2 * m
      start_load(j0 + 1, buf1, sem1)
      wait_load(j0, buf0, sem0)
      scatter(j0, buf0)
      start_load(j0 + 2, buf0, sem0)
      wait_load(j0 + 1, buf1, sem1)
      scatter(j0 + 1, buf1)
      return 0
    lax.fori_loop(0, npairs, pair, 0)
    wait_load(lo + 2 * npairs, buf0, sem0)

  def finalize(base, span):
    stripe = t * (span // NS)
    for off, sz in _stripe_subs(span):
      pltpu.sync_copy(acc_sh.at[pl.ds(stripe + off, sz), :],
                      sums_buf.at[pl.ds(0, sz), :])
      pltpu.sync_copy(cnt_sh.at[pl.ds(stripe + off, sz), :],
                      cnts_buf.at[pl.ds(0, sz), :])

      def per_seg(i, _):
        cvec = cnts_buf[i, :]
        r = 1.0 / jnp.maximum(cvec, 1.0)
        for k in range(HALF // 16):
          sums_buf[i, pl.ds(k * 16, 16)] = sums_buf[i, pl.ds(k * 16, 16)] * r
        return 0
      lax.fori_loop(0, sz, per_seg, 0)

      pltpu.sync_copy(sums_buf.at[pl.ds(0, sz), :],
                      out_hbm.at[pl.ds(base + stripe + off, sz),
                                 pl.ds(col0, HALF)])

  def refill(i, _):
    for k in range(HALF // 16):
      sums_buf[i, pl.ds(k * 16, 16)] = zeros16
    cnts_buf[i, :] = zeros16
    return 0

  for p in range(3):
    run_pass(p, pass_lo[p], pass_hi[p])
    plsc.subcore_barrier()
    finalize(BASES[p], SPANS[p])
    if p < 2:
      lax.fori_loop(0, 128, refill, 0)
      zero_acc()
      plsc.subcore_barrier()


def _build():
  return pl.kernel(
      _body,
      out_type=jax.ShapeDtypeStruct((S_PAD, D), jnp.float32),
      mesh=plsc.VectorSubcoreMesh(
          core_axis_name="c", subcore_axis_name="s",
          num_cores=NC, num_subcores=NS),
      scratch_types=[
          pltpu.VMEM((16,), jnp.int32),
          pltpu.VMEM((NTPT + 8, CH), jnp.int32),
          pltpu.VMEM((CH, HALF), jnp.float32),
          pltpu.VMEM((CH, HALF), jnp.float32),
          pltpu.VMEM((128, HALF), jnp.float32),
          pltpu.VMEM((128, 16), jnp.float32),
          pltpu.VMEM((CH, 16), jnp.float32),
          pltpu.VMEM_SHARED((ACC_ROWS + 8, HALF), jnp.float32),
          pltpu.VMEM_SHARED((ACC_ROWS + 8, 16), jnp.float32),
          pltpu.SemaphoreType.DMA,
          pltpu.SemaphoreType.DMA,
      ],
      compiler_params=pltpu.CompilerParams(use_tc_tiling_on_sc=False),
  )


def _tileize(a):
  return a.reshape(NCH_PAD // NS, NS, CH).transpose(1, 0, 2)


@jax.jit
def kernel(x, segment_ids):
  ids = jnp.asarray(segment_ids, jnp.int32)
  ids = jnp.concatenate(
      [ids, jnp.full((NCH_PAD * CH - ROWS,), S - 1, jnp.int32)])
  idx = jnp.stack([
      _tileize(jnp.where((ids >= b) & (ids < b + sp), ids - b, DUMMY))
      for b, sp in zip(BASES, SPANS)])
  idx = jnp.concatenate(
      [idx, jnp.full((3, NS, 8, CH), DUMMY, jnp.int32)], axis=2)
  idx = idx.reshape(3 * NS, NTPT + 8, CH)
  firsts = ids[0::CH].reshape(NCH_PAD // NS, NS)
  lasts = ids[CH - 1::CH].reshape(NCH_PAD // NS, NS)
  jj = jnp.arange(NCH_PAD // NS)[:, None]
  tt = jnp.arange(NS)[None, :]
  valid = (tt + NS * jj) < NCHUNKS
  nt = valid.sum(axis=0).astype(jnp.int32)
  hi0 = (valid & (firsts < BASES[1])).sum(axis=0).astype(jnp.int32)
  hi1 = (valid & (firsts < BASES[2])).sum(axis=0).astype(jnp.int32)
  lo1 = (valid & (lasts < BASES[1])).sum(axis=0).astype(jnp.int32)
  lo2 = (valid & (lasts < BASES[2])).sum(axis=0).astype(jnp.int32)
  bounds = jnp.stack(
      [hi0, hi1, lo1, lo2, nt] + [jnp.zeros((NS,), jnp.int32)] * 11,
      axis=1)
  return _build()(x, idx, bounds)[:S]
